# Initial kernel scaffold; baseline (speedup 1.0000x reference)
import jax
import jax.numpy as jnp
from jax.experimental import pallas as pl


def kernel(x1, edge_index1, edge_attr1, batch1, x2, edge_index2, edge_attr2, batch2, W0_root, W0_rel, b0, W1, b1, W_att0, W_att1, W_m1, b_m1, W_m2, b_m2):
    raise NotImplementedError("write your pallas kernel here")


# speedup vs baseline: 5.0520x; 5.0520x over previous
import functools

import jax
import jax.numpy as jnp
from jax import lax
from jax.experimental import pallas as pl
from jax.experimental.pallas import tpu as pltpu
from jax.experimental.pallas import tpu_sc as plsc

N = 10000
E = 320000
D = 128
H = 64
B = 64

NP = N + 48
T = 2 * NP
WR = 80
NC = 2
NS = 16
NW = NC * NS
EPT = 655360
EPW = EPT // NW
PAD = EPT - 2 * E
RPT = T // NS


def _sc_body(gw, sw, scale, ch, *refs):
    if scale:
        (table, gidx, scidx2, ew, zrows,
         orows, gidxf_v, scidx_v, ew_v, rows_v, wrows_v, acc, sem) = refs
    else:
        (table, gidx, scidx2, zrows,
         orows, gidxf_v, scidx_v, rows_v, acc, sem) = refs
        wrows_v = rows_v
    nb = ch // 128

    c = lax.axis_index("c")
    s = lax.axis_index("s")
    w = s * NC + c

    pltpu.sync_copy(zrows.at[pl.ds(s * RPT, RPT)], acc.at[pl.ds(s * RPT, RPT)])
    plsc.subcore_barrier()

    i16 = lax.broadcasted_iota(jnp.int32, (16,), 0)
    m0 = i16 == 0
    onev = jnp.where(i16 == 1, 1.0, 0.0).astype(jnp.float32)

    def chunk(ci, carry):
        pltpu.sync_copy(gidx.at[w, pl.ds(ci * ch, ch)], gidxf_v)
        pltpu.sync_copy(scidx2.at[w, pl.ds(ci * nb, nb)], scidx_v)
        if scale:
            pltpu.sync_copy(ew.at[w, pl.ds(ci * ch, ch)], ew_v)
        cps = []
        for b in range(nb):
            cps.append(pltpu.async_copy(
                table.at[gidxf_v.at[pl.ds(b * 128, 128)]],
                rows_v.at[pl.ds(b * 128, 128)], sem))
        for cp in cps:
            cp.wait()
        if scale:
            def grp(g, cy):
                base = g * 16
                ewg = ew_v[pl.ds(base, 16)]
                for e in range(16):
                    i = base + e
                    wv = ewg[e]
                    for j in range(gw // 16):
                        wrows_v[i, pl.ds(j * 16, 16)] = rows_v[i, pl.ds(j * 16, 16)] * wv
                    wrows_v[i, pl.ds(gw, 16)] = jnp.where(m0, wv, onev)
                return cy
            lax.fori_loop(0, ch // 16, grp, 0)
        for b in range(nb):
            pltpu.sync_copy(wrows_v.at[pl.ds(b * 128, 128)],
                            acc.at[scidx_v.at[b]], add=True)
        return carry

    lax.fori_loop(0, EPW // ch, chunk, 0)
    plsc.subcore_barrier()
    pltpu.sync_copy(acc.at[pl.ds(s * RPT, RPT)], orows.at[c, pl.ds(s * RPT, RPT)])


def _make_sc_pass(gw, sw, scale, ch):
    nb = ch // 128
    mesh = plsc.VectorSubcoreMesh(core_axis_name="c", subcore_axis_name="s")
    outs = jax.ShapeDtypeStruct((NC, T, sw), jnp.float32)
    scratch = [pltpu.VMEM((ch,), jnp.int32),
               pltpu.VMEM((nb, 128), jnp.int32)]
    if scale:
        scratch.append(pltpu.VMEM((ch,), jnp.float32))
    scratch.append(pltpu.VMEM((ch, gw), jnp.float32))
    if scale:
        scratch.append(pltpu.VMEM((ch, sw), jnp.float32))
    scratch.append(pltpu.VMEM_SHARED((T, sw), jnp.float32))
    scratch.append(pltpu.SemaphoreType.DMA)
    return pl.kernel(functools.partial(_sc_body, gw, sw, scale, ch),
                     out_type=outs, mesh=mesh, scratch_types=scratch,
                     compiler_params=pltpu.CompilerParams(use_tc_tiling_on_sc=False),
                     name=f"sc_edge_pass_{gw}_{sw}_{int(scale)}")


_sc_pass_A = _make_sc_pass(H, WR, True, 128)
_sc_pass_B = _make_sc_pass(H, H, False, 512)
_sc_pass_C = _make_sc_pass(WR, WR, False, 256)


def _tc1_body(x_ref, w_ref, o_ref):
    o_ref[...] = jnp.dot(x_ref[...], w_ref[...], preferred_element_type=jnp.float32)


def _leaky(x):
    return jnp.where(x > 0, x, 0.2 * x)


def _readout(f, batch2d, w_att):
    onehot = (batch2d == lax.broadcasted_iota(jnp.int32, (B, NP), 0)).astype(jnp.float32)
    cntb = jnp.sum(onehot, axis=1)[:, None]
    mean = jnp.dot(onehot, f, preferred_element_type=jnp.float32) / jnp.maximum(cntb, 1.0)
    t = jnp.tanh(jnp.dot(mean, w_att, preferred_element_type=jnp.float32))
    gathered = lax.dot_general(onehot, t, (((0,), (0,)), ((), ())),
                               preferred_element_type=jnp.float32)
    gate = jax.nn.sigmoid(jnp.sum(f * gathered, axis=1, keepdims=True))
    return jnp.dot(onehot, gate * f, preferred_element_type=jnp.float32)


def _tc2_body(xroot_ref, rows2_ref, b0_ref, w1_ref, batch_ref, watt_ref,
              xt_ref, hw_ref, de_ref, att_ref):
    agg = rows2_ref[0, :, :H] + rows2_ref[1, :, :H]
    deg = (rows2_ref[0, :, H] + rows2_ref[1, :, H])[:, None]
    cnt = (rows2_ref[0, :, H + 1] + rows2_ref[1, :, H + 1])[:, None]
    f0 = _leaky(xroot_ref[...] + agg / (deg + 1e-6) + b0_ref[...])
    xt_ref[...] = jnp.dot(f0, w1_ref[...], preferred_element_type=jnp.float32)
    hw_ref[...] = (deg + 1.0) / (cnt + 1.0)
    de_ref[...] = cnt + 1.0
    att_ref[...] = _readout(f0, batch_ref[...], watt_ref[...])


def _tc3_body(maggs_ref, xt_ref, hw_ref, de_ref, mhw_ref):
    magg = maggs_ref[0] + maggs_ref[1]
    m = (magg + xt_ref[...]) / (de_ref[...] + 1e-6)
    mhw_ref[:, :H] = m * hw_ref[...]
    mhw_ref[:, H:] = jnp.concatenate(
        [hw_ref[...], jnp.zeros((NP, WR - H - 1), jnp.float32)], axis=1)


def _tc4_body(crows_ref, mhw_ref, b1_ref, batch_ref, watt_ref, att_ref):
    oagg = crows_ref[0, :, :H] + crows_ref[1, :, :H]
    dvp = (crows_ref[0, :, H] + crows_ref[1, :, H])[:, None]
    mh = mhw_ref[:, :H]
    hw = mhw_ref[:, H][:, None]
    dv = dvp + hw
    out = (oagg + mh) / (dv + 1e-6) + b1_ref[...]
    f1 = _leaky(out)
    att_ref[...] = _readout(f1, batch_ref[...], watt_ref[...])


def _tc5_body(a10_ref, a20_ref, a11_ref, a21_ref, wm1_ref, bm1_ref, wm2_ref, bm2_ref,
              scores_ref, sim_ref):
    scores = jnp.concatenate([a10_ref[...], a20_ref[...], a11_ref[...], a21_ref[...]],
                             axis=1)
    scores_ref[...] = scores
    h = jax.nn.relu(jnp.dot(scores, wm1_ref[...], preferred_element_type=jnp.float32)
                    + bm1_ref[...])
    sim_ref[...] = jax.nn.sigmoid(jnp.dot(h, wm2_ref[...], preferred_element_type=jnp.float32)
                                  + bm2_ref[...])


_tc1 = pl.pallas_call(_tc1_body, out_shape=jax.ShapeDtypeStruct((2 * NP, 2 * H), jnp.float32))
_tc2 = pl.pallas_call(_tc2_body, out_shape=[
    jax.ShapeDtypeStruct((NP, H), jnp.float32),
    jax.ShapeDtypeStruct((NP, 1), jnp.float32),
    jax.ShapeDtypeStruct((NP, 1), jnp.float32),
    jax.ShapeDtypeStruct((B, H), jnp.float32),
])
_tc3 = pl.pallas_call(_tc3_body, out_shape=jax.ShapeDtypeStruct((NP, WR), jnp.float32))
_tc4 = pl.pallas_call(_tc4_body, out_shape=jax.ShapeDtypeStruct((B, H), jnp.float32))
_tc5 = pl.pallas_call(_tc5_body, out_shape=[
    jax.ShapeDtypeStruct((B, 4 * H), jnp.float32),
    jax.ShapeDtypeStruct((B, 1), jnp.float32),
])


@jax.jit
def _impl(x1, edge_index1, edge_attr1, batch1, x2, edge_index2, edge_attr2, batch2,
          W0_root, W0_rel, b0, W1, b1, W_att0, W_att1, W_m1, b_m1, W_m2, b_m2):
    f32 = jnp.float32
    xs = jnp.concatenate([
        jnp.pad(x1, ((0, NP - N), (0, 0))), jnp.pad(x2, ((0, NP - N), (0, 0)))], axis=0)
    srcc = jnp.concatenate([edge_index1[0], edge_index2[0] + NP,
                            jnp.full((PAD,), N, jnp.int32)])
    dstc = jnp.concatenate([edge_index1[1], edge_index2[1] + NP,
                            jnp.full((PAD,), N, jnp.int32)])
    ewc = jnp.concatenate([edge_attr1, edge_attr2, jnp.zeros((PAD,), f32)])
    src_f = srcc.reshape(NW, EPW)
    dst_f = dstc.reshape(NW, EPW)
    src_2 = srcc.reshape(NW, EPW // 128, 128)
    dst_2 = dstc.reshape(NW, EPW // 128, 128)
    ew_f = ewc.reshape(NW, EPW)
    zrows_w = jnp.zeros((T, WR), f32)
    zrows_h = jnp.zeros((T, H), f32)
    batch1p = jnp.concatenate([batch1, jnp.full((NP - N,), B, jnp.int32)])[None, :]
    batch2p = jnp.concatenate([batch2, jnp.full((NP - N,), B, jnp.int32)])[None, :]

    wcat = jnp.concatenate([W0_root, W0_rel], axis=1)
    xrcat = _tc1(xs, wcat)
    xr_table = xrcat[:, H:]
    xroot1 = xrcat[:NP, :H]
    xroot2 = xrcat[NP:, :H]

    arows = _sc_pass_A(xr_table, src_f, dst_2, ew_f, zrows_w)

    b0r = b0[None, :]
    xt1, hw1, de1, att1_0 = _tc2(xroot1, arows[:, :NP], b0r, W1, batch1p, W_att0)
    xt2, hw2, de2, att2_0 = _tc2(xroot2, arows[:, NP:], b0r, W1, batch2p, W_att0)

    xt_table = jnp.concatenate([xt1, xt2], axis=0)
    brows = _sc_pass_B(xt_table, src_f, dst_2, zrows_h)

    mhw1 = _tc3(brows[:, :NP], xt1, hw1, de1)
    mhw2 = _tc3(brows[:, NP:], xt2, hw2, de2)

    mhw_table = jnp.concatenate([mhw1, mhw2], axis=0)
    crows = _sc_pass_C(mhw_table, dst_f, src_2, zrows_w)

    b1r = b1[None, :]
    att1_1 = _tc4(crows[:, :NP], mhw1, b1r, batch1p, W_att1)
    att2_1 = _tc4(crows[:, NP:], mhw2, b1r, batch2p, W_att1)

    scores, sim = _tc5(att1_0, att2_0, att1_1, att2_1,
                       W_m1, b_m1[None, :], W_m2, b_m2[None, :])
    return (scores, sim)


def kernel(x1, edge_index1, edge_attr1, batch1, x2, edge_index2, edge_attr2, batch2,
           W0_root, W0_rel, b0, W1, b1, W_att0, W_att1, W_m1, b_m1, W_m2, b_m2):
    return _impl(x1, edge_index1, edge_attr1, batch1,
                 x2, edge_index2, edge_attr2, batch2,
                 W0_root, W0_rel, b0, W1, b1, W_att0, W_att1, W_m1, b_m1, W_m2, b_m2)


# speedup vs baseline: 6.2430x; 1.2357x over previous
import functools

import jax
import jax.numpy as jnp
from jax import lax
from jax.experimental import pallas as pl
from jax.experimental.pallas import tpu as pltpu
from jax.experimental.pallas import tpu_sc as plsc

N = 10000
E = 320000
D = 128
H = 64
B = 64

NP = N + 112
T = 2 * NP
WR = 80
NC = 2
NS = 16
NW = NC * NS
EPG = 10240
PADG = NW * EPG - E
RPG = NP // NS


def _sc_body(gw, sw, scale, ch, *refs):
    if scale:
        (table, gidx, scidx2, ew, zrows, orows,
         gidxf0, gidxf1, scidx0, scidx1, ew0, ew1,
         rows0, rows1, wrows, acc, sem0, sem1) = refs
    else:
        (table, gidx, scidx2, zrows, orows,
         gidxf0, gidxf1, scidx0, scidx1,
         rows0, rows1, acc, sem0, sem1) = refs
        ew0 = ew1 = wrows = None
    nb = ch // 128
    npg = EPG // ch

    c = lax.axis_index("c")
    s = lax.axis_index("s")
    w = s * NC + c

    i16 = lax.broadcasted_iota(jnp.int32, (16,), 0)
    m0 = i16 == 0
    onev = jnp.where(i16 == 1, 1.0, 0.0).astype(jnp.float32)

    def stage_and_fire(g, cidx, gidxf_v, scidx_v, ew_v, rows_v, sem):
        pltpu.sync_copy(gidx.at[g, w, pl.ds(cidx * ch, ch)], gidxf_v)
        pltpu.sync_copy(scidx2.at[g, w, pl.ds(cidx * nb, nb)], scidx_v)
        if scale:
            pltpu.sync_copy(ew.at[g, w, pl.ds(cidx * ch, ch)], ew_v)
        for b in range(nb):
            pltpu.async_copy(table.at[gidxf_v.at[pl.ds(b * 128, 128)]],
                             rows_v.at[pl.ds(b * 128, 128)], sem)

    def drain(rows_v, sem):
        pltpu.make_async_copy(table.at[pl.ds(0, ch)], rows_v, sem).wait()

    def process_and_scatter(rows_v, scidx_v, ew_v):
        if scale:
            def grp(gg, cy):
                base = gg * 16
                ewg = ew_v[pl.ds(base, 16)]
                for e in range(16):
                    i = base + e
                    wv = ewg[e]
                    for j in range(gw // 16):
                        wrows[i, pl.ds(j * 16, 16)] = rows_v[i, pl.ds(j * 16, 16)] * wv
                    wrows[i, pl.ds(gw, 16)] = jnp.where(m0, wv, onev)
                return cy
            lax.fori_loop(0, ch // 16, grp, 0)
            srcv = wrows
        else:
            srcv = rows_v
        for b in range(nb):
            pltpu.sync_copy(srcv.at[pl.ds(b * 128, 128)],
                            acc.at[scidx_v.at[b]], add=True)

    for g in range(2):
        pltpu.sync_copy(zrows.at[pl.ds(s * RPG, RPG)], acc.at[pl.ds(s * RPG, RPG)])
        plsc.subcore_barrier()
        stage_and_fire(g, 0, gidxf0, scidx0, ew0, rows0, sem0)

        def pair(p, cy):
            c1 = 2 * p + 1
            c2 = lax.rem(2 * p + 2, npg)
            drain(rows0, sem0)
            stage_and_fire(g, c1, gidxf1, scidx1, ew1, rows1, sem1)
            process_and_scatter(rows0, scidx0, ew0)
            drain(rows1, sem1)
            stage_and_fire(g, c2, gidxf0, scidx0, ew0, rows0, sem0)
            process_and_scatter(rows1, scidx1, ew1)
            return cy

        lax.fori_loop(0, npg // 2, pair, 0)
        drain(rows0, sem0)
        plsc.subcore_barrier()
        pltpu.sync_copy(acc.at[pl.ds(s * RPG, RPG)],
                        orows.at[c, g, pl.ds(s * RPG, RPG)])


def _make_sc_pass(gw, sw, scale, ch):
    nb = ch // 128
    mesh = plsc.VectorSubcoreMesh(core_axis_name="c", subcore_axis_name="s")
    outs = jax.ShapeDtypeStruct((NC, 2, NP, sw), jnp.float32)
    scratch = [pltpu.VMEM((ch,), jnp.int32), pltpu.VMEM((ch,), jnp.int32),
               pltpu.VMEM((nb, 128), jnp.int32), pltpu.VMEM((nb, 128), jnp.int32)]
    if scale:
        scratch += [pltpu.VMEM((ch,), jnp.float32), pltpu.VMEM((ch,), jnp.float32)]
    scratch += [pltpu.VMEM((ch, gw), jnp.float32), pltpu.VMEM((ch, gw), jnp.float32)]
    if scale:
        scratch.append(pltpu.VMEM((ch, sw), jnp.float32))
    scratch.append(pltpu.VMEM_SHARED((NP, sw), jnp.float32))
    scratch += [pltpu.SemaphoreType.DMA, pltpu.SemaphoreType.DMA]
    return pl.kernel(functools.partial(_sc_body, gw, sw, scale, ch),
                     out_type=outs, mesh=mesh, scratch_types=scratch,
                     compiler_params=pltpu.CompilerParams(use_tc_tiling_on_sc=False),
                     name=f"sc_edge_pass_{gw}_{sw}_{int(scale)}")


_sc_pass_A = _make_sc_pass(H, WR, True, 256)
_sc_pass_B = _make_sc_pass(H, H, False, 512)
_sc_pass_C = _make_sc_pass(WR, WR, False, 256)


def _tc1_body(x_ref, w_ref, o_ref):
    o_ref[...] = jnp.dot(x_ref[...], w_ref[...], preferred_element_type=jnp.float32)


def _leaky(x):
    return jnp.where(x > 0, x, 0.2 * x)


def _readout(f, batch2d, w_att):
    onehot = (batch2d == lax.broadcasted_iota(jnp.int32, (B, NP), 0)).astype(jnp.float32)
    cntb = jnp.sum(onehot, axis=1)[:, None]
    mean = jnp.dot(onehot, f, preferred_element_type=jnp.float32) / jnp.maximum(cntb, 1.0)
    t = jnp.tanh(jnp.dot(mean, w_att, preferred_element_type=jnp.float32))
    gathered = lax.dot_general(onehot, t, (((0,), (0,)), ((), ())),
                               preferred_element_type=jnp.float32)
    gate = jax.nn.sigmoid(jnp.sum(f * gathered, axis=1, keepdims=True))
    return jnp.dot(onehot, gate * f, preferred_element_type=jnp.float32)


def _tc2_body(xroot_ref, rows2_ref, b0_ref, w1_ref, batch_ref, watt_ref,
              xt_ref, hw_ref, de_ref, att_ref):
    agg = rows2_ref[0, :, :H] + rows2_ref[1, :, :H]
    deg = (rows2_ref[0, :, H] + rows2_ref[1, :, H])[:, None]
    cnt = (rows2_ref[0, :, H + 1] + rows2_ref[1, :, H + 1])[:, None]
    f0 = _leaky(xroot_ref[...] + agg / (deg + 1e-6) + b0_ref[...])
    xt_ref[...] = jnp.dot(f0, w1_ref[...], preferred_element_type=jnp.float32)
    hw_ref[...] = (deg + 1.0) / (cnt + 1.0)
    de_ref[...] = cnt + 1.0
    att_ref[...] = _readout(f0, batch_ref[...], watt_ref[...])


def _tc3_body(maggs_ref, xt_ref, hw_ref, de_ref, mhw_ref):
    magg = maggs_ref[0] + maggs_ref[1]
    m = (magg + xt_ref[...]) / (de_ref[...] + 1e-6)
    mhw_ref[:, :H] = m * hw_ref[...]
    mhw_ref[:, H:] = jnp.concatenate(
        [hw_ref[...], jnp.zeros((NP, WR - H - 1), jnp.float32)], axis=1)


def _tc4_body(crows_ref, mhw_ref, b1_ref, batch_ref, watt_ref, att_ref):
    oagg = crows_ref[0, :, :H] + crows_ref[1, :, :H]
    dvp = (crows_ref[0, :, H] + crows_ref[1, :, H])[:, None]
    mh = mhw_ref[:, :H]
    hw = mhw_ref[:, H][:, None]
    dv = dvp + hw
    out = (oagg + mh) / (dv + 1e-6) + b1_ref[...]
    f1 = _leaky(out)
    att_ref[...] = _readout(f1, batch_ref[...], watt_ref[...])


def _tc5_body(a10_ref, a20_ref, a11_ref, a21_ref, wm1_ref, bm1_ref, wm2_ref, bm2_ref,
              scores_ref, sim_ref):
    scores = jnp.concatenate([a10_ref[...], a20_ref[...], a11_ref[...], a21_ref[...]],
                             axis=1)
    scores_ref[...] = scores
    h = jax.nn.relu(jnp.dot(scores, wm1_ref[...], preferred_element_type=jnp.float32)
                    + bm1_ref[...])
    sim_ref[...] = jax.nn.sigmoid(jnp.dot(h, wm2_ref[...], preferred_element_type=jnp.float32)
                                  + bm2_ref[...])


_tc1 = pl.pallas_call(_tc1_body, out_shape=jax.ShapeDtypeStruct((2 * NP, 2 * H), jnp.float32))
_tc2 = pl.pallas_call(_tc2_body, out_shape=[
    jax.ShapeDtypeStruct((NP, H), jnp.float32),
    jax.ShapeDtypeStruct((NP, 1), jnp.float32),
    jax.ShapeDtypeStruct((NP, 1), jnp.float32),
    jax.ShapeDtypeStruct((B, H), jnp.float32),
])
_tc3 = pl.pallas_call(_tc3_body, out_shape=jax.ShapeDtypeStruct((NP, WR), jnp.float32))
_tc4 = pl.pallas_call(_tc4_body, out_shape=jax.ShapeDtypeStruct((B, H), jnp.float32))
_tc5 = pl.pallas_call(_tc5_body, out_shape=[
    jax.ShapeDtypeStruct((B, 4 * H), jnp.float32),
    jax.ShapeDtypeStruct((B, 1), jnp.float32),
])


@jax.jit
def _impl(x1, edge_index1, edge_attr1, batch1, x2, edge_index2, edge_attr2, batch2,
          W0_root, W0_rel, b0, W1, b1, W_att0, W_att1, W_m1, b_m1, W_m2, b_m2):
    f32 = jnp.float32
    xs = jnp.concatenate([
        jnp.pad(x1, ((0, NP - N), (0, 0))), jnp.pad(x2, ((0, NP - N), (0, 0)))], axis=0)
    sinks = jnp.full((PADG,), N, jnp.int32)
    s1 = jnp.concatenate([edge_index1[0], sinks]).reshape(NW, EPG)
    s2 = jnp.concatenate([edge_index2[0], sinks]).reshape(NW, EPG)
    d1 = jnp.concatenate([edge_index1[1], sinks]).reshape(NW, EPG)
    d2 = jnp.concatenate([edge_index2[1], sinks]).reshape(NW, EPG)
    zpad = jnp.zeros((PADG,), f32)
    ew1 = jnp.concatenate([edge_attr1, zpad]).reshape(NW, EPG)
    ew2 = jnp.concatenate([edge_attr2, zpad]).reshape(NW, EPG)
    gsrc = jnp.stack([s1, s2 + NP])
    gdst = jnp.stack([d1, d2 + NP])
    sdst = jnp.stack([d1, d2]).reshape(2, NW, EPG // 128, 128)
    ssrc = jnp.stack([s1, s2]).reshape(2, NW, EPG // 128, 128)
    ew_st = jnp.stack([ew1, ew2])
    zrows_w = jnp.zeros((NP, WR), f32)
    zrows_h = jnp.zeros((NP, H), f32)
    batch1p = jnp.concatenate([batch1, jnp.full((NP - N,), B, jnp.int32)])[None, :]
    batch2p = jnp.concatenate([batch2, jnp.full((NP - N,), B, jnp.int32)])[None, :]

    wcat = jnp.concatenate([W0_root, W0_rel], axis=1)
    xrcat = _tc1(xs, wcat)
    xr_table = xrcat[:, H:]
    xroot1 = xrcat[:NP, :H]
    xroot2 = xrcat[NP:, :H]

    arows = _sc_pass_A(xr_table, gsrc, sdst, ew_st, zrows_w)

    b0r = b0[None, :]
    xt1, hw1, de1, att1_0 = _tc2(xroot1, arows[:, 0], b0r, W1, batch1p, W_att0)
    xt2, hw2, de2, att2_0 = _tc2(xroot2, arows[:, 1], b0r, W1, batch2p, W_att0)

    xt_table = jnp.concatenate([xt1, xt2], axis=0)
    brows = _sc_pass_B(xt_table, gsrc, sdst, zrows_h)

    mhw1 = _tc3(brows[:, 0], xt1, hw1, de1)
    mhw2 = _tc3(brows[:, 1], xt2, hw2, de2)

    mhw_table = jnp.concatenate([mhw1, mhw2], axis=0)
    crows = _sc_pass_C(mhw_table, gdst, ssrc, zrows_w)

    b1r = b1[None, :]
    att1_1 = _tc4(crows[:, 0], mhw1, b1r, batch1p, W_att1)
    att2_1 = _tc4(crows[:, 1], mhw2, b1r, batch2p, W_att1)

    scores, sim = _tc5(att1_0, att2_0, att1_1, att2_1,
                       W_m1, b_m1[None, :], W_m2, b_m2[None, :])
    return (scores, sim)


def kernel(x1, edge_index1, edge_attr1, batch1, x2, edge_index2, edge_attr2, batch2,
           W0_root, W0_rel, b0, W1, b1, W_att0, W_att1, W_m1, b_m1, W_m2, b_m2):
    return _impl(x1, edge_index1, edge_attr1, batch1,
                 x2, edge_index2, edge_attr2, batch2,
                 W0_root, W0_rel, b0, W1, b1, W_att0, W_att1, W_m1, b_m1, W_m2, b_m2)


# speedup vs baseline: 7.4706x; 1.1966x over previous
import functools

import jax
import jax.numpy as jnp
from jax import lax
from jax.experimental import pallas as pl
from jax.experimental.pallas import tpu as pltpu
from jax.experimental.pallas import tpu_sc as plsc

N = 10000
E = 320000
D = 128
H = 64
B = 64

NP = N + 112
T = 2 * NP
WR = 80
NC = 2
NS = 16
NW = NC * NS
EG = 327680
PADG = EG - E
RPG = NP // NS


def _sc_body(gw, sw, scale, ch, k0, k1, *refs):
    if scale:
        (table, gidx, scidx2, ew, zrows, orows,
         gidxf0, gidxf1, scidx0, scidx1, ew0, ew1,
         rows0, rows1, wrows, acc, sem0, sem1) = refs
    else:
        (table, gidx, scidx2, zrows, orows,
         gidxf0, gidxf1, scidx0, scidx1,
         rows0, rows1, acc, sem0, sem1) = refs
        ew0 = ew1 = wrows = None
    nb = ch // 128

    c = lax.axis_index("c")
    s = lax.axis_index("s")
    nch = k0 + c * (k1 - k0)
    cbase = c * NS * k0 + s * nch

    i16 = lax.broadcasted_iota(jnp.int32, (16,), 0)
    m0 = i16 == 0
    onev = jnp.where(i16 == 1, 1.0, 0.0).astype(jnp.float32)

    if scale:
        pltpu.sync_copy(zrows.at[pl.ds(0, ch)], wrows)

    def stage_and_fire(g, cidx, gidxf_v, scidx_v, ew_v, rows_v, sem):
        gc = cbase + cidx
        pltpu.sync_copy(gidx.at[g, pl.ds(gc * ch, ch)], gidxf_v)
        pltpu.sync_copy(scidx2.at[g, pl.ds(gc * nb, nb)], scidx_v)
        if scale:
            pltpu.sync_copy(ew.at[g, pl.ds(gc * ch, ch)], ew_v)
        for b in range(nb):
            pltpu.async_copy(table.at[gidxf_v.at[pl.ds(b * 128, 128)]],
                             rows_v.at[pl.ds(b * 128, 128)], sem)

    def drain(rows_v, sem):
        pltpu.make_async_copy(table.at[pl.ds(0, ch)], rows_v, sem).wait()

    def process_and_scatter(rows_v, scidx_v, ew_v):
        if scale:
            for gg in range(ch // 16):
                base = gg * 16
                ewg = ew_v[pl.ds(base, 16)]
                for e in range(16):
                    i = base + e
                    wv = ewg[e]
                    for j in range(gw // 16):
                        wrows[i, pl.ds(j * 16, 16)] = rows_v[i, pl.ds(j * 16, 16)] * wv
                    wrows[i, pl.ds(gw, 16)] = jnp.where(m0, wv, onev)
            srcv = wrows
        else:
            srcv = rows_v
        for b in range(nb):
            pltpu.sync_copy(srcv.at[pl.ds(b * 128, 128)],
                            acc.at[scidx_v.at[b]], add=True)

    for g in range(2):
        pltpu.sync_copy(zrows.at[pl.ds(s * RPG, RPG)], acc.at[pl.ds(s * RPG, RPG)])
        plsc.subcore_barrier()
        stage_and_fire(g, 0, gidxf0, scidx0, ew0, rows0, sem0)

        def pair(p, cy):
            c1 = 2 * p + 1
            c2 = lax.rem(2 * p + 2, nch)
            drain(rows0, sem0)
            stage_and_fire(g, c1, gidxf1, scidx1, ew1, rows1, sem1)
            process_and_scatter(rows0, scidx0, ew0)
            drain(rows1, sem1)
            stage_and_fire(g, c2, gidxf0, scidx0, ew0, rows0, sem0)
            process_and_scatter(rows1, scidx1, ew1)
            return cy

        lax.fori_loop(0, nch // 2, pair, 0)
        drain(rows0, sem0)
        plsc.subcore_barrier()
        pltpu.sync_copy(acc.at[pl.ds(s * RPG, RPG)],
                        orows.at[c, g, pl.ds(s * RPG, RPG)])


def _make_sc_pass(gw, sw, scale, ch, k0, k1):
    nb = ch // 128
    assert NS * (k0 + k1) * ch == EG and k0 % 2 == 0 and k1 % 2 == 0
    mesh = plsc.VectorSubcoreMesh(core_axis_name="c", subcore_axis_name="s")
    outs = jax.ShapeDtypeStruct((NC, 2, NP, sw), jnp.float32)
    scratch = [pltpu.VMEM((ch,), jnp.int32), pltpu.VMEM((ch,), jnp.int32),
               pltpu.VMEM((nb, 128), jnp.int32), pltpu.VMEM((nb, 128), jnp.int32)]
    if scale:
        scratch += [pltpu.VMEM((ch,), jnp.float32), pltpu.VMEM((ch,), jnp.float32)]
    scratch += [pltpu.VMEM((ch, gw), jnp.float32), pltpu.VMEM((ch, gw), jnp.float32)]
    if scale:
        scratch.append(pltpu.VMEM((ch, sw), jnp.float32))
    scratch.append(pltpu.VMEM_SHARED((NP, sw), jnp.float32))
    scratch += [pltpu.SemaphoreType.DMA, pltpu.SemaphoreType.DMA]
    return pl.kernel(functools.partial(_sc_body, gw, sw, scale, ch, k0, k1),
                     out_type=outs, mesh=mesh, scratch_types=scratch,
                     compiler_params=pltpu.CompilerParams(use_tc_tiling_on_sc=False),
                     name=f"sc_edge_pass_{gw}_{sw}_{int(scale)}")


_sc_pass_A = _make_sc_pass(H, WR, True, 128, 104, 56)
_sc_pass_B = _make_sc_pass(H, H, False, 512, 28, 12)
_sc_pass_C = _make_sc_pass(WR, WR, False, 256, 56, 24)


def _tc1_body(x_ref, w_ref, o_ref):
    o_ref[...] = jnp.dot(x_ref[...], w_ref[...], preferred_element_type=jnp.float32)


def _leaky(x):
    return jnp.where(x > 0, x, 0.2 * x)


def _readout(f, batch2d, w_att):
    onehot = (batch2d == lax.broadcasted_iota(jnp.int32, (B, NP), 0)).astype(jnp.float32)
    cntb = jnp.sum(onehot, axis=1)[:, None]
    mean = jnp.dot(onehot, f, preferred_element_type=jnp.float32) / jnp.maximum(cntb, 1.0)
    t = jnp.tanh(jnp.dot(mean, w_att, preferred_element_type=jnp.float32))
    gathered = lax.dot_general(onehot, t, (((0,), (0,)), ((), ())),
                               preferred_element_type=jnp.float32)
    gate = jax.nn.sigmoid(jnp.sum(f * gathered, axis=1, keepdims=True))
    return jnp.dot(onehot, gate * f, preferred_element_type=jnp.float32)


def _tc2_body(xroot_ref, rows2_ref, b0_ref, w1_ref, batch_ref, watt_ref,
              xt_ref, hw_ref, de_ref, att_ref):
    agg = rows2_ref[0, :, :H] + rows2_ref[1, :, :H]
    deg = (rows2_ref[0, :, H] + rows2_ref[1, :, H])[:, None]
    cnt = (rows2_ref[0, :, H + 1] + rows2_ref[1, :, H + 1])[:, None]
    f0 = _leaky(xroot_ref[...] + agg / (deg + 1e-6) + b0_ref[...])
    xt_ref[...] = jnp.dot(f0, w1_ref[...], preferred_element_type=jnp.float32)
    hw_ref[...] = (deg + 1.0) / (cnt + 1.0)
    de_ref[...] = cnt + 1.0
    att_ref[...] = _readout(f0, batch_ref[...], watt_ref[...])


def _tc3_body(maggs_ref, xt_ref, hw_ref, de_ref, mhw_ref):
    magg = maggs_ref[0] + maggs_ref[1]
    m = (magg + xt_ref[...]) / (de_ref[...] + 1e-6)
    mhw_ref[:, :H] = m * hw_ref[...]
    mhw_ref[:, H:] = jnp.concatenate(
        [hw_ref[...], jnp.zeros((NP, WR - H - 1), jnp.float32)], axis=1)


def _tc4_body(crows_ref, mhw_ref, b1_ref, batch_ref, watt_ref, att_ref):
    oagg = crows_ref[0, :, :H] + crows_ref[1, :, :H]
    dvp = (crows_ref[0, :, H] + crows_ref[1, :, H])[:, None]
    mh = mhw_ref[:, :H]
    hw = mhw_ref[:, H][:, None]
    dv = dvp + hw
    out = (oagg + mh) / (dv + 1e-6) + b1_ref[...]
    f1 = _leaky(out)
    att_ref[...] = _readout(f1, batch_ref[...], watt_ref[...])


def _tc5_body(a10_ref, a20_ref, a11_ref, a21_ref, wm1_ref, bm1_ref, wm2_ref, bm2_ref,
              scores_ref, sim_ref):
    scores = jnp.concatenate([a10_ref[...], a20_ref[...], a11_ref[...], a21_ref[...]],
                             axis=1)
    scores_ref[...] = scores
    h = jax.nn.relu(jnp.dot(scores, wm1_ref[...], preferred_element_type=jnp.float32)
                    + bm1_ref[...])
    sim_ref[...] = jax.nn.sigmoid(jnp.dot(h, wm2_ref[...], preferred_element_type=jnp.float32)
                                  + bm2_ref[...])


_tc1 = pl.pallas_call(_tc1_body, out_shape=jax.ShapeDtypeStruct((2 * NP, 2 * H), jnp.float32))
_tc2 = pl.pallas_call(_tc2_body, out_shape=[
    jax.ShapeDtypeStruct((NP, H), jnp.float32),
    jax.ShapeDtypeStruct((NP, 1), jnp.float32),
    jax.ShapeDtypeStruct((NP, 1), jnp.float32),
    jax.ShapeDtypeStruct((B, H), jnp.float32),
])
_tc3 = pl.pallas_call(_tc3_body, out_shape=jax.ShapeDtypeStruct((NP, WR), jnp.float32))
_tc4 = pl.pallas_call(_tc4_body, out_shape=jax.ShapeDtypeStruct((B, H), jnp.float32))
_tc5 = pl.pallas_call(_tc5_body, out_shape=[
    jax.ShapeDtypeStruct((B, 4 * H), jnp.float32),
    jax.ShapeDtypeStruct((B, 1), jnp.float32),
])


@jax.jit
def _impl(x1, edge_index1, edge_attr1, batch1, x2, edge_index2, edge_attr2, batch2,
          W0_root, W0_rel, b0, W1, b1, W_att0, W_att1, W_m1, b_m1, W_m2, b_m2):
    f32 = jnp.float32
    xs = jnp.concatenate([
        jnp.pad(x1, ((0, NP - N), (0, 0))), jnp.pad(x2, ((0, NP - N), (0, 0)))], axis=0)
    sinks = jnp.full((PADG,), N, jnp.int32)
    s1 = jnp.concatenate([edge_index1[0], sinks])
    s2 = jnp.concatenate([edge_index2[0], sinks])
    d1 = jnp.concatenate([edge_index1[1], sinks])
    d2 = jnp.concatenate([edge_index2[1], sinks])
    zpad = jnp.zeros((PADG,), f32)
    ew1 = jnp.concatenate([edge_attr1, zpad])
    ew2 = jnp.concatenate([edge_attr2, zpad])
    gsrc = jnp.stack([s1, s2 + NP])
    gdst = jnp.stack([d1, d2 + NP])
    sdst = jnp.stack([d1, d2]).reshape(2, EG // 128, 128)
    ssrc = jnp.stack([s1, s2]).reshape(2, EG // 128, 128)
    ew_st = jnp.stack([ew1, ew2])
    zrows_w = jnp.zeros((NP, WR), f32)
    zrows_h = jnp.zeros((NP, H), f32)
    batch1p = jnp.concatenate([batch1, jnp.full((NP - N,), B, jnp.int32)])[None, :]
    batch2p = jnp.concatenate([batch2, jnp.full((NP - N,), B, jnp.int32)])[None, :]

    wcat = jnp.concatenate([W0_root, W0_rel], axis=1)
    xrcat = _tc1(xs, wcat)
    xr_table = xrcat[:, H:]
    xroot1 = xrcat[:NP, :H]
    xroot2 = xrcat[NP:, :H]

    arows = _sc_pass_A(xr_table, gsrc, sdst, ew_st, zrows_w)

    b0r = b0[None, :]
    xt1, hw1, de1, att1_0 = _tc2(xroot1, arows[:, 0], b0r, W1, batch1p, W_att0)
    xt2, hw2, de2, att2_0 = _tc2(xroot2, arows[:, 1], b0r, W1, batch2p, W_att0)

    xt_table = jnp.concatenate([xt1, xt2], axis=0)
    brows = _sc_pass_B(xt_table, gsrc, sdst, zrows_h)

    mhw1 = _tc3(brows[:, 0], xt1, hw1, de1)
    mhw2 = _tc3(brows[:, 1], xt2, hw2, de2)

    mhw_table = jnp.concatenate([mhw1, mhw2], axis=0)
    crows = _sc_pass_C(mhw_table, gdst, ssrc, zrows_w)

    b1r = b1[None, :]
    att1_1 = _tc4(crows[:, 0], mhw1, b1r, batch1p, W_att1)
    att2_1 = _tc4(crows[:, 1], mhw2, b1r, batch2p, W_att1)

    scores, sim = _tc5(att1_0, att2_0, att1_1, att2_1,
                       W_m1, b_m1[None, :], W_m2, b_m2[None, :])
    return (scores, sim)


def kernel(x1, edge_index1, edge_attr1, batch1, x2, edge_index2, edge_attr2, batch2,
           W0_root, W0_rel, b0, W1, b1, W_att0, W_att1, W_m1, b_m1, W_m2, b_m2):
    return _impl(x1, edge_index1, edge_attr1, batch1,
                 x2, edge_index2, edge_attr2, batch2,
                 W0_root, W0_rel, b0, W1, b1, W_att0, W_att1, W_m1, b_m1, W_m2, b_m2)
